# Initial kernel scaffold; baseline (speedup 1.0000x reference)
import jax
import jax.numpy as jnp
from jax.experimental import pallas as pl


def kernel(atom_features, pair_features, pair_split, atom_to_pair, W_pa, b_pa, W_ao, b_ao, W_aa, b_aa, W_ap, b_ap, W_pp, b_pp):
    raise NotImplementedError("write your pallas kernel here")


# speedup vs baseline: 1.7831x; 1.7831x over previous
import functools

import jax
import jax.numpy as jnp
from jax import lax
from jax.experimental import pallas as pl
from jax.experimental.pallas import tpu as pltpu
from jax.experimental.pallas import tpu_sc as plsc

N = 50000
E = 800000
DA = 75
DP = 16
DG = 32
DO = 50
DXAP = 64
DXC = 96

NC = 2
NS = 16
NW = NC * NS
BC = 128
NCH0 = 195
EXTRA = (E - NW * NCH0 * BC) // BC
NPAD = 50048
ROWS_PER_TILE = NPAD // NS
ZROWS = 184
ZREP = ROWS_PER_TILE // ZROWS


def _node_pre_body(a_ref, wpa_ref, wap_ref, waot_ref, bao_ref, waa_ref, baa_ref,
                   xc_ref, xap_ref, pre_ref, aaa_ref):
    a = a_ref[...]
    xap = jnp.dot(a, wap_ref[...], preferred_element_type=jnp.float32)
    xpa = jnp.dot(a, wpa_ref[...], preferred_element_type=jnp.float32)
    pad = jnp.zeros((a.shape[0], DXAP - DO), jnp.float32)
    xc_ref[...] = jnp.concatenate([xap, pad, xpa], axis=1)
    xap_ref[...] = jnp.concatenate([xap, pad], axis=1)
    pre_ref[...] = jnp.dot(a, waot_ref[...], preferred_element_type=jnp.float32) + bao_ref[...]
    aaa_ref[...] = jnp.maximum(jnp.dot(a, waa_ref[...], preferred_element_type=jnp.float32) + baa_ref[...], 0.0)


def _node_pre(atom, wpa_a, wap_a, wao_top, b_ao, waa, b_aa):
    bm = 2000
    grid = (N // bm,)
    return pl.pallas_call(
        _node_pre_body,
        grid=grid,
        in_specs=[
            pl.BlockSpec((bm, DA), lambda i: (i, 0)),
            pl.BlockSpec((DA, DG), lambda i: (0, 0)),
            pl.BlockSpec((DA, DO), lambda i: (0, 0)),
            pl.BlockSpec((DA, DO), lambda i: (0, 0)),
            pl.BlockSpec((1, DO), lambda i: (0, 0)),
            pl.BlockSpec((DA, DO), lambda i: (0, 0)),
            pl.BlockSpec((1, DO), lambda i: (0, 0)),
        ],
        out_specs=[
            pl.BlockSpec((bm, DXC), lambda i: (i, 0)),
            pl.BlockSpec((bm, DXAP), lambda i: (i, 0)),
            pl.BlockSpec((bm, DO), lambda i: (i, 0)),
            pl.BlockSpec((bm, DO), lambda i: (i, 0)),
        ],
        out_shape=[
            jax.ShapeDtypeStruct((N, DXC), jnp.float32),
            jax.ShapeDtypeStruct((N, DXAP), jnp.float32),
            jax.ShapeDtypeStruct((N, DO), jnp.float32),
            jax.ShapeDtypeStruct((N, DO), jnp.float32),
        ],
    )(atom, wpa_a, wap_a, wao_top, b_ao, waa, b_aa)


def _edge_pre_body(pf_ref, w_ref, b_ref, u_ref):
    u_ref[...] = jnp.dot(pf_ref[...], w_ref[...], preferred_element_type=jnp.float32) + b_ref[...]


def _edge_pre(pf, wpa_p, b_pa):
    bm = 8000
    return pl.pallas_call(
        _edge_pre_body,
        grid=(E // bm,),
        in_specs=[
            pl.BlockSpec((bm, DP), lambda i: (i, 0)),
            pl.BlockSpec((DP, DG), lambda i: (0, 0)),
            pl.BlockSpec((1, DG), lambda i: (0, 0)),
        ],
        out_specs=pl.BlockSpec((bm, DG), lambda i: (i, 0)),
        out_shape=jax.ShapeDtypeStruct((E, DG), jnp.float32),
    )(pf, wpa_p, b_pa)


def _sc_edge_body(xc_hbm, xap_hbm, u_hbm, pj_hbm, pi_hbm,
                  s_hbm, g2_hbm,
                  idxj_v, idxi_v, xcj_v, vi_v, u_v, zbuf_v,
                  accum, semA, semB):
    c = lax.axis_index("c")
    s = lax.axis_index("s")
    wid = c * NS + s
    base = wid * (NCH0 * BC) + jnp.minimum(wid, EXTRA) * BC
    nch = NCH0 + jnp.where(wid < EXTRA, 1, 0)

    zero16 = jnp.zeros((16,), jnp.float32)

    def zrow(r, carry):
        zbuf_v[r, pl.ds(0, 16)] = zero16
        zbuf_v[r, pl.ds(16, 16)] = zero16
        return carry

    lax.fori_loop(0, ZROWS, zrow, 0)
    for k in range(ZREP):
        pltpu.sync_copy(zbuf_v, accum.at[pl.ds(s * ROWS_PER_TILE + k * ZROWS, ZROWS)])
    plsc.subcore_barrier()

    def chunk(t, carry):
        eb = base + t * BC
        pltpu.sync_copy(pj_hbm.at[pl.ds(eb, BC)], idxj_v)
        pltpu.sync_copy(pi_hbm.at[pl.ds(eb, BC)], idxi_v)
        cpA = pltpu.async_copy(xc_hbm.at[idxj_v], xcj_v, semA)
        cpB = pltpu.async_copy(xap_hbm.at[idxi_v], vi_v, semB)
        pltpu.sync_copy(u_hbm.at[pl.ds(eb, BC)], u_v)
        cpA.wait()
        cpB.wait()

        def row(r, rc):
            for cc in range(4):
                sl = pl.ds(cc * 16, 16)
                vi_v[r, sl] = vi_v[r, sl] + xcj_v[r, sl]
            for cc in range(2):
                slo = pl.ds(cc * 16, 16)
                sli = pl.ds(DXAP + cc * 16, 16)
                u_v[r, slo] = jnp.maximum(u_v[r, slo] + xcj_v[r, sli], 0.0)
            return rc

        lax.fori_loop(0, BC, row, 0)
        pltpu.sync_copy(vi_v, g2_hbm.at[pl.ds(eb, BC)])
        pltpu.sync_copy(u_v, accum.at[idxi_v], add=True)
        return carry

    lax.fori_loop(0, nch, chunk, 0)
    plsc.subcore_barrier()
    pltpu.sync_copy(accum.at[pl.ds(s * ROWS_PER_TILE, ROWS_PER_TILE)],
                    s_hbm.at[pl.ds(c * NPAD + s * ROWS_PER_TILE, ROWS_PER_TILE)])


def _sc_edge(xc, xap64, u, pj, pi):
    mesh = plsc.VectorSubcoreMesh(core_axis_name="c", subcore_axis_name="s")
    fn = functools.partial(
        pl.kernel,
        mesh=mesh,
        compiler_params=pltpu.CompilerParams(use_tc_tiling_on_sc=False),
        out_type=[
            jax.ShapeDtypeStruct((NC * NPAD, DG), jnp.float32),
            jax.ShapeDtypeStruct((E, DXAP), jnp.float32),
        ],
        scratch_types=[
            pltpu.VMEM((BC,), jnp.int32),
            pltpu.VMEM((BC,), jnp.int32),
            pltpu.VMEM((BC, DXC), jnp.float32),
            pltpu.VMEM((BC, DXAP), jnp.float32),
            pltpu.VMEM((BC, DG), jnp.float32),
            pltpu.VMEM((ZROWS, DG), jnp.float32),
            pltpu.VMEM_SHARED((NPAD, DG), jnp.float32),
            pltpu.SemaphoreType.DMA,
            pltpu.SemaphoreType.DMA,
        ],
    )(_sc_edge_body)
    return fn(xc, xap64, u, pj, pi)


def _atom_post_body(s0_ref, s1_ref, pre_ref, aaa_ref, w_ref, out_ref):
    seg = s0_ref[...] + s1_ref[...]
    a_pa = jnp.maximum(pre_ref[...] + jnp.dot(seg, w_ref[...], preferred_element_type=jnp.float32), 0.0)
    out_ref[...] = jnp.maximum(a_pa + aaa_ref[...], 0.0)


def _atom_post(s0, s1, pre, aaa, wao_agg):
    bm = 2000
    return pl.pallas_call(
        _atom_post_body,
        grid=(N // bm,),
        in_specs=[
            pl.BlockSpec((bm, DG), lambda i: (i, 0)),
            pl.BlockSpec((bm, DG), lambda i: (i, 0)),
            pl.BlockSpec((bm, DO), lambda i: (i, 0)),
            pl.BlockSpec((bm, DO), lambda i: (i, 0)),
            pl.BlockSpec((DG, DO), lambda i: (0, 0)),
        ],
        out_specs=pl.BlockSpec((bm, DO), lambda i: (i, 0)),
        out_shape=jax.ShapeDtypeStruct((N, DO), jnp.float32),
    )(s0, s1, pre, aaa, wao_agg)


def _pair_post_body(pf_ref, g2_ref, wap_ref, bap_ref, wpp_ref, bpp_ref, out_ref):
    pf = pf_ref[...]
    papa = jnp.maximum(
        jnp.dot(pf, wap_ref[...], preferred_element_type=jnp.float32)
        + g2_ref[:, :DO] + bap_ref[...], 0.0)
    ppp = jnp.maximum(
        jnp.dot(pf, wpp_ref[...], preferred_element_type=jnp.float32) + bpp_ref[...], 0.0)
    out_ref[...] = jnp.maximum(papa + ppp, 0.0)


def _pair_post(pf, g2, wap_p, b_ap, wpp, b_pp):
    bm = 8000
    return pl.pallas_call(
        _pair_post_body,
        grid=(E // bm,),
        in_specs=[
            pl.BlockSpec((bm, DP), lambda i: (i, 0)),
            pl.BlockSpec((bm, DXAP), lambda i: (i, 0)),
            pl.BlockSpec((DP, DO), lambda i: (0, 0)),
            pl.BlockSpec((1, DO), lambda i: (0, 0)),
            pl.BlockSpec((DP, DO), lambda i: (0, 0)),
            pl.BlockSpec((1, DO), lambda i: (0, 0)),
        ],
        out_specs=pl.BlockSpec((bm, DO), lambda i: (i, 0)),
        out_shape=jax.ShapeDtypeStruct((E, DO), jnp.float32),
    )(pf, g2, wap_p, b_ap, wpp, b_pp)


def kernel(atom_features, pair_features, pair_split, atom_to_pair,
           W_pa, b_pa, W_ao, b_ao, W_aa, b_aa, W_ap, b_ap, W_pp, b_pp):
    pi = atom_to_pair[:, 0]
    pj = atom_to_pair[:, 1]

    wpa_p, wpa_a = W_pa[:DP], W_pa[DP:]
    wap_p, wap_a = W_ap[:DP], W_ap[DP:]
    wao_top, wao_agg = W_ao[:DA], W_ao[DA:]

    b_pa2 = b_pa.reshape(1, DG)
    b_ao2 = b_ao.reshape(1, DO)
    b_aa2 = b_aa.reshape(1, DO)
    b_ap2 = b_ap.reshape(1, DO)
    b_pp2 = b_pp.reshape(1, DO)

    xc, xap64, pre, aaa = _node_pre(atom_features, wpa_a, wap_a, wao_top, b_ao2, waa=W_aa, b_aa=b_aa2)
    u = _edge_pre(pair_features, wpa_p, b_pa2)
    s_all, g2 = _sc_edge(xc, xap64, u, pj, pi)
    s0 = lax.slice(s_all, (0, 0), (N, DG))
    s1 = lax.slice(s_all, (NPAD, 0), (NPAD + N, DG))
    atom_hidden = _atom_post(s0, s1, pre, aaa, wao_agg)
    pair_hidden = _pair_post(pair_features, g2, wap_p, b_ap2, wpp=W_pp, b_pp=b_pp2)
    return (atom_hidden, pair_hidden)


# speedup vs baseline: 2.2781x; 1.2776x over previous
import functools

import jax
import jax.numpy as jnp
from jax import lax
from jax.experimental import pallas as pl
from jax.experimental.pallas import tpu as pltpu
from jax.experimental.pallas import tpu_sc as plsc

N = 50000
E = 800000
DA = 75
DP = 16
DG = 32
DO = 50
DXAP = 64
DXC = 96

NC = 2
NS = 16
NW = NC * NS
EPW = E // NW
BC = 200
NCH = EPW // BC
SCB = 40
NSC = BC // SCB
NPAD = 50048
ROWS_PER_TILE = NPAD // NS
ZROWS = 184
ZREP = ROWS_PER_TILE // ZROWS


def _node_pre_body(a_ref, wpa_ref, wap_ref, waot_ref, bao_ref, waa_ref, baa_ref,
                   xpa_ref, xap_ref, pre_ref, aaa_ref):
    a = a_ref[...]
    xap = jnp.dot(a, wap_ref[...], preferred_element_type=jnp.float32)
    xpa_ref[...] = jnp.dot(a, wpa_ref[...], preferred_element_type=jnp.float32)
    pad = jnp.zeros((a.shape[0], DXAP - DO), jnp.float32)
    xap_ref[...] = jnp.concatenate([xap, pad], axis=1)
    pre_ref[...] = jnp.dot(a, waot_ref[...], preferred_element_type=jnp.float32) + bao_ref[...]
    aaa_ref[...] = jnp.maximum(jnp.dot(a, waa_ref[...], preferred_element_type=jnp.float32) + baa_ref[...], 0.0)


def _node_pre(atom, wpa_a, wap_a, wao_top, b_ao, waa, b_aa):
    bm = 2000
    grid = (N // bm,)
    return pl.pallas_call(
        _node_pre_body,
        grid=grid,
        in_specs=[
            pl.BlockSpec((bm, DA), lambda i: (i, 0)),
            pl.BlockSpec((DA, DG), lambda i: (0, 0)),
            pl.BlockSpec((DA, DO), lambda i: (0, 0)),
            pl.BlockSpec((DA, DO), lambda i: (0, 0)),
            pl.BlockSpec((1, DO), lambda i: (0, 0)),
            pl.BlockSpec((DA, DO), lambda i: (0, 0)),
            pl.BlockSpec((1, DO), lambda i: (0, 0)),
        ],
        out_specs=[
            pl.BlockSpec((bm, DG), lambda i: (i, 0)),
            pl.BlockSpec((bm, DXAP), lambda i: (i, 0)),
            pl.BlockSpec((bm, DO), lambda i: (i, 0)),
            pl.BlockSpec((bm, DO), lambda i: (i, 0)),
        ],
        out_shape=[
            jax.ShapeDtypeStruct((N, DG), jnp.float32),
            jax.ShapeDtypeStruct((N, DXAP), jnp.float32),
            jax.ShapeDtypeStruct((N, DO), jnp.float32),
            jax.ShapeDtypeStruct((N, DO), jnp.float32),
        ],
    )(atom, wpa_a, wap_a, wao_top, b_ao, waa, b_aa)


def _edge_pre_body(pf_ref, w_ref, b_ref, u_ref):
    u_ref[...] = jnp.dot(pf_ref[...], w_ref[...], preferred_element_type=jnp.float32) + b_ref[...]


def _edge_pre(pf, wpa_p, b_pa):
    bm = 8000
    return pl.pallas_call(
        _edge_pre_body,
        grid=(E // bm,),
        in_specs=[
            pl.BlockSpec((bm, DP), lambda i: (i, 0)),
            pl.BlockSpec((DP, DG), lambda i: (0, 0)),
            pl.BlockSpec((1, DG), lambda i: (0, 0)),
        ],
        out_specs=pl.BlockSpec((bm, DG), lambda i: (i, 0)),
        out_shape=jax.ShapeDtypeStruct((E, DG), jnp.float32),
    )(pf, wpa_p, b_pa)


def _sc_pair_body(xap_hbm, pj_hbm, pi_hbm, g2_hbm,
                  idxj_v, idxi_v, vj_v, vi_v, g2b_v, semA, semB, semC):
    c = lax.axis_index("c")
    s = lax.axis_index("s")
    wid = c * NS + s
    base = wid * EPW

    def stage_a(t, b):
        eb = base + t * BC
        pltpu.sync_copy(pj_hbm.at[pl.ds(eb, BC)], idxj_v[b])
        pltpu.sync_copy(pi_hbm.at[pl.ds(eb, BC)], idxi_v[b])
        pltpu.async_copy(xap_hbm.at[idxj_v[b]], vj_v[b], semA[b])
        pltpu.async_copy(xap_hbm.at[idxi_v[b]], vi_v[b], semB[b])

    def drain_a(b):
        pltpu.make_async_copy(xap_hbm.at[idxj_v[b]], vj_v[b], semA[b]).wait()
        pltpu.make_async_copy(xap_hbm.at[idxi_v[b]], vi_v[b], semB[b]).wait()

    def drain_c(b):
        pltpu.make_async_copy(g2b_v[b], g2_hbm.at[pl.ds(0, BC)], semC[b]).wait()

    def stage_b(t, b, not_first):
        eb = base + t * BC
        drain_a(b)

        @pl.when(not_first)
        def _():
            drain_c(b)

        def row(r2, rc):
            for rr in range(2):
                r = r2 * 2 + rr
                for cc in range(4):
                    sl = pl.ds(cc * 16, 16)
                    g2b_v[b][r, sl] = vi_v[b][r, sl] + vj_v[b][r, sl]
            return rc

        lax.fori_loop(0, BC // 2, row, 0)
        pltpu.async_copy(g2b_v[b], g2_hbm.at[pl.ds(eb, BC)], semC[b])

    stage_a(0, 0)
    stage_a(1, 1)

    def pair(q, carry):
        t0 = 2 * q
        stage_b(t0, 0, q > 0)
        stage_a(t0 + 2, 0)
        stage_b(t0 + 1, 1, q > 0)

        @pl.when(q < (NCH - 1) // 2 - 1)
        def _():
            stage_a(t0 + 3, 1)

        return carry

    lax.fori_loop(0, (NCH - 1) // 2, pair, 0)
    stage_b(NCH - 1, 0, jnp.bool_(True))
    drain_c(0)
    drain_c(1)


def _sc_pair(xap64, pj, pi):
    mesh = plsc.VectorSubcoreMesh(core_axis_name="c", subcore_axis_name="s")
    fn = functools.partial(
        pl.kernel,
        mesh=mesh,
        compiler_params=pltpu.CompilerParams(use_tc_tiling_on_sc=False),
        out_type=jax.ShapeDtypeStruct((E, DXAP), jnp.float32),
        scratch_types=[
            [pltpu.VMEM((BC,), jnp.int32)] * 2,
            [pltpu.VMEM((BC,), jnp.int32)] * 2,
            [pltpu.VMEM((BC, DXAP), jnp.float32)] * 2,
            [pltpu.VMEM((BC, DXAP), jnp.float32)] * 2,
            [pltpu.VMEM((BC, DXAP), jnp.float32)] * 2,
            [pltpu.SemaphoreType.DMA] * 2,
            [pltpu.SemaphoreType.DMA] * 2,
            [pltpu.SemaphoreType.DMA] * 2,
        ],
    )(_sc_pair_body)
    return fn(xap64, pj, pi)


def _sc_atom_body(xpa_hbm, u_hbm, pj_hbm, pi_hbm, zeros_hbm,
                  s_hbm,
                  idxj_v, idxis_v, xpj_v, u_v, accum, semA, semE):
    c = lax.axis_index("c")
    s = lax.axis_index("s")
    wid = c * NS + s
    base = wid * EPW

    pltpu.sync_copy(zeros_hbm.at[pl.ds(s * ROWS_PER_TILE, ROWS_PER_TILE)],
                    accum.at[pl.ds(s * ROWS_PER_TILE, ROWS_PER_TILE)])
    plsc.subcore_barrier()

    def stage_a(t, b):
        eb = base + t * BC
        pltpu.sync_copy(pj_hbm.at[pl.ds(eb, BC)], idxj_v[b])
        pltpu.async_copy(xpa_hbm.at[idxj_v[b]], xpj_v[b], semA[b])
        for k in range(NSC):
            pltpu.async_copy(pi_hbm.at[pl.ds(eb + k * SCB, SCB)], idxis_v[b].at[k], semE[b])
        pltpu.async_copy(u_hbm.at[pl.ds(eb, BC)], u_v[b], semE[b])

    def drain_a(b):
        pltpu.make_async_copy(xpa_hbm.at[idxj_v[b]], xpj_v[b], semA[b]).wait()
        for k in range(NSC):
            pltpu.make_async_copy(pi_hbm.at[pl.ds(0, SCB)], idxis_v[b].at[k], semE[b]).wait()
        pltpu.make_async_copy(u_hbm.at[pl.ds(0, BC)], u_v[b], semE[b]).wait()

    def stage_b(t, b):
        drain_a(b)

        def row(r2, rc):
            for rr in range(2):
                r = r2 * 2 + rr
                for cc in range(2):
                    sl = pl.ds(cc * 16, 16)
                    u_v[b][r, sl] = jnp.maximum(u_v[b][r, sl] + xpj_v[b][r, sl], 0.0)
            return rc

        lax.fori_loop(0, BC // 2, row, 0)
        for k in range(NSC):
            pltpu.sync_copy(u_v[b].at[pl.ds(k * SCB, SCB)],
                            accum.at[idxis_v[b].at[k]], add=True)

    stage_a(0, 0)
    stage_a(1, 1)

    def pair(q, carry):
        t0 = 2 * q
        stage_b(t0, 0)
        stage_a(t0 + 2, 0)
        stage_b(t0 + 1, 1)

        @pl.when(q < (NCH - 1) // 2 - 1)
        def _():
            stage_a(t0 + 3, 1)

        return carry

    lax.fori_loop(0, (NCH - 1) // 2, pair, 0)
    stage_b(NCH - 1, 0)

    plsc.subcore_barrier()
    pltpu.sync_copy(accum.at[pl.ds(s * ROWS_PER_TILE, ROWS_PER_TILE)],
                    s_hbm.at[pl.ds(c * NPAD + s * ROWS_PER_TILE, ROWS_PER_TILE)])


def _sc_atom(xpa, u, pj, pi, zeros):
    mesh = plsc.VectorSubcoreMesh(core_axis_name="c", subcore_axis_name="s")
    fn = functools.partial(
        pl.kernel,
        mesh=mesh,
        compiler_params=pltpu.CompilerParams(use_tc_tiling_on_sc=False),
        out_type=jax.ShapeDtypeStruct((NC * NPAD, DG), jnp.float32),
        scratch_types=[
            [pltpu.VMEM((BC,), jnp.int32)] * 2,
            [pltpu.VMEM((NSC, SCB), jnp.int32)] * 2,
            [pltpu.VMEM((BC, DG), jnp.float32)] * 2,
            [pltpu.VMEM((BC, DG), jnp.float32)] * 2,
            pltpu.VMEM_SHARED((NPAD, DG), jnp.float32),
            [pltpu.SemaphoreType.DMA] * 2,
            [pltpu.SemaphoreType.DMA] * 2,
        ],
    )(_sc_atom_body)
    return fn(xpa, u, pj, pi, zeros)


def _atom_post_body(s0_ref, s1_ref, pre_ref, aaa_ref, w_ref, out_ref):
    seg = s0_ref[...] + s1_ref[...]
    a_pa = jnp.maximum(pre_ref[...] + jnp.dot(seg, w_ref[...], preferred_element_type=jnp.float32), 0.0)
    out_ref[...] = jnp.maximum(a_pa + aaa_ref[...], 0.0)


def _atom_post(s0, s1, pre, aaa, wao_agg):
    bm = 2000
    return pl.pallas_call(
        _atom_post_body,
        grid=(N // bm,),
        in_specs=[
            pl.BlockSpec((bm, DG), lambda i: (i, 0)),
            pl.BlockSpec((bm, DG), lambda i: (i, 0)),
            pl.BlockSpec((bm, DO), lambda i: (i, 0)),
            pl.BlockSpec((bm, DO), lambda i: (i, 0)),
            pl.BlockSpec((DG, DO), lambda i: (0, 0)),
        ],
        out_specs=pl.BlockSpec((bm, DO), lambda i: (i, 0)),
        out_shape=jax.ShapeDtypeStruct((N, DO), jnp.float32),
    )(s0, s1, pre, aaa, wao_agg)


def _pair_post_body(pf_ref, g2_ref, wap_ref, bap_ref, wpp_ref, bpp_ref, out_ref):
    pf = pf_ref[...]
    papa = jnp.maximum(
        jnp.dot(pf, wap_ref[...], preferred_element_type=jnp.float32)
        + g2_ref[:, :DO] + bap_ref[...], 0.0)
    ppp = jnp.maximum(
        jnp.dot(pf, wpp_ref[...], preferred_element_type=jnp.float32) + bpp_ref[...], 0.0)
    out_ref[...] = jnp.maximum(papa + ppp, 0.0)


def _pair_post(pf, g2, wap_p, b_ap, wpp, b_pp):
    bm = 8000
    return pl.pallas_call(
        _pair_post_body,
        grid=(E // bm,),
        in_specs=[
            pl.BlockSpec((bm, DP), lambda i: (i, 0)),
            pl.BlockSpec((bm, DXAP), lambda i: (i, 0)),
            pl.BlockSpec((DP, DO), lambda i: (0, 0)),
            pl.BlockSpec((1, DO), lambda i: (0, 0)),
            pl.BlockSpec((DP, DO), lambda i: (0, 0)),
            pl.BlockSpec((1, DO), lambda i: (0, 0)),
        ],
        out_specs=pl.BlockSpec((bm, DO), lambda i: (i, 0)),
        out_shape=jax.ShapeDtypeStruct((E, DO), jnp.float32),
    )(pf, g2, wap_p, b_ap, wpp, b_pp)


def kernel(atom_features, pair_features, pair_split, atom_to_pair,
           W_pa, b_pa, W_ao, b_ao, W_aa, b_aa, W_ap, b_ap, W_pp, b_pp):
    pi = atom_to_pair[:, 0]
    pj = atom_to_pair[:, 1]

    wpa_p, wpa_a = W_pa[:DP], W_pa[DP:]
    wap_p, wap_a = W_ap[:DP], W_ap[DP:]
    wao_top, wao_agg = W_ao[:DA], W_ao[DA:]

    b_pa2 = b_pa.reshape(1, DG)
    b_ao2 = b_ao.reshape(1, DO)
    b_aa2 = b_aa.reshape(1, DO)
    b_ap2 = b_ap.reshape(1, DO)
    b_pp2 = b_pp.reshape(1, DO)

    xpa, xap64, pre, aaa = _node_pre(atom_features, wpa_a, wap_a, wao_top, b_ao2, waa=W_aa, b_aa=b_aa2)
    u = _edge_pre(pair_features, wpa_p, b_pa2)
    zeros = jnp.zeros((NPAD, DG), jnp.float32)
    g2 = _sc_pair(xap64, pj, pi)
    s_all = _sc_atom(xpa, u, pj, pi, zeros)
    s0 = lax.slice(s_all, (0, 0), (N, DG))
    s1 = lax.slice(s_all, (NPAD, 0), (NPAD + N, DG))
    atom_hidden = _atom_post(s0, s1, pre, aaa, wao_agg)
    pair_hidden = _pair_post(pair_features, g2, wap_p, b_ap2, wpp=W_pp, b_pp=b_pp2)
    return (atom_hidden, pair_hidden)


# speedup vs baseline: 2.3583x; 1.0352x over previous
import functools

import jax
import jax.numpy as jnp
from jax import lax
from jax.experimental import pallas as pl
from jax.experimental.pallas import tpu as pltpu
from jax.experimental.pallas import tpu_sc as plsc

N = 50000
E = 800000
DA = 75
DP = 16
DG = 32
DO = 50
DXAP = 64
DXC = 96

NC = 2
NS = 16
NW = NC * NS
EPW = E // NW
BC = 200
NCH = EPW // BC
SCB = 40
NSC = BC // SCB
NPAD = 50048
ROWS_PER_TILE = NPAD // NS
ZROWS = 184
ZREP = ROWS_PER_TILE // ZROWS


def _node_pre_body(a_ref, wpa_ref, wap_ref, waot_ref, bao_ref, waa_ref, baa_ref,
                   xpa_ref, xap_ref, pre_ref, aaa_ref):
    a = a_ref[...]
    xap = jnp.dot(a, wap_ref[...], preferred_element_type=jnp.float32)
    xpa_ref[...] = jnp.dot(a, wpa_ref[...], preferred_element_type=jnp.float32)
    pad = jnp.zeros((a.shape[0], DXAP - DO), jnp.float32)
    xap_ref[...] = jnp.concatenate([xap, pad], axis=1).astype(jnp.bfloat16)
    pre_ref[...] = jnp.dot(a, waot_ref[...], preferred_element_type=jnp.float32) + bao_ref[...]
    aaa_ref[...] = jnp.maximum(jnp.dot(a, waa_ref[...], preferred_element_type=jnp.float32) + baa_ref[...], 0.0)


def _node_pre(atom, wpa_a, wap_a, wao_top, b_ao, waa, b_aa):
    bm = 2000
    grid = (N // bm,)
    return pl.pallas_call(
        _node_pre_body,
        grid=grid,
        in_specs=[
            pl.BlockSpec((bm, DA), lambda i: (i, 0)),
            pl.BlockSpec((DA, DG), lambda i: (0, 0)),
            pl.BlockSpec((DA, DO), lambda i: (0, 0)),
            pl.BlockSpec((DA, DO), lambda i: (0, 0)),
            pl.BlockSpec((1, DO), lambda i: (0, 0)),
            pl.BlockSpec((DA, DO), lambda i: (0, 0)),
            pl.BlockSpec((1, DO), lambda i: (0, 0)),
        ],
        out_specs=[
            pl.BlockSpec((bm, DG), lambda i: (i, 0)),
            pl.BlockSpec((bm, DXAP), lambda i: (i, 0)),
            pl.BlockSpec((bm, DO), lambda i: (i, 0)),
            pl.BlockSpec((bm, DO), lambda i: (i, 0)),
        ],
        out_shape=[
            jax.ShapeDtypeStruct((N, DG), jnp.float32),
            jax.ShapeDtypeStruct((N, DXAP), jnp.bfloat16),
            jax.ShapeDtypeStruct((N, DO), jnp.float32),
            jax.ShapeDtypeStruct((N, DO), jnp.float32),
        ],
    )(atom, wpa_a, wap_a, wao_top, b_ao, waa, b_aa)


def _edge_pre_body(pf_ref, w_ref, b_ref, u_ref):
    u_ref[...] = jnp.dot(pf_ref[...], w_ref[...], preferred_element_type=jnp.float32) + b_ref[...]


def _edge_pre(pf, wpa_p, b_pa):
    bm = 8000
    return pl.pallas_call(
        _edge_pre_body,
        grid=(E // bm,),
        in_specs=[
            pl.BlockSpec((bm, DP), lambda i: (i, 0)),
            pl.BlockSpec((DP, DG), lambda i: (0, 0)),
            pl.BlockSpec((1, DG), lambda i: (0, 0)),
        ],
        out_specs=pl.BlockSpec((bm, DG), lambda i: (i, 0)),
        out_shape=jax.ShapeDtypeStruct((E, DG), jnp.float32),
    )(pf, wpa_p, b_pa)


def _sc_pair_body(xap_hbm, pj_hbm, pi_hbm, g2_hbm,
                  idxj_v, idxi_v, vj_v, vi_v, g2b_v, semA, semB, semC):
    c = lax.axis_index("c")
    s = lax.axis_index("s")
    wid = c * NS + s
    base = wid * EPW

    def stage_a(t, b):
        eb = base + t * BC
        pltpu.sync_copy(pj_hbm.at[pl.ds(eb, BC)], idxj_v[b])
        pltpu.sync_copy(pi_hbm.at[pl.ds(eb, BC)], idxi_v[b])
        pltpu.async_copy(xap_hbm.at[idxj_v[b]], vj_v[b], semA[b])
        pltpu.async_copy(xap_hbm.at[idxi_v[b]], vi_v[b], semB[b])

    def drain_a(b):
        pltpu.make_async_copy(xap_hbm.at[idxj_v[b]], vj_v[b], semA[b]).wait()
        pltpu.make_async_copy(xap_hbm.at[idxi_v[b]], vi_v[b], semB[b]).wait()

    def drain_c(b):
        pltpu.make_async_copy(g2b_v[b], g2_hbm.at[pl.ds(0, BC)], semC[b]).wait()

    def stage_b(t, b, not_first):
        eb = base + t * BC
        drain_a(b)

        @pl.when(not_first)
        def _():
            drain_c(b)

        def row(r2, rc):
            for rr in range(2):
                r = r2 * 2 + rr
                for cc in range(2):
                    sl = pl.ds(cc * 32, 32)
                    g2b_v[b][r, sl] = vi_v[b][r, sl] + vj_v[b][r, sl]
            return rc

        lax.fori_loop(0, BC // 2, row, 0)
        pltpu.async_copy(g2b_v[b], g2_hbm.at[pl.ds(eb, BC)], semC[b])

    stage_a(0, 0)
    stage_a(1, 1)

    def pair(q, carry):
        t0 = 2 * q
        stage_b(t0, 0, q > 0)
        stage_a(t0 + 2, 0)
        stage_b(t0 + 1, 1, q > 0)

        @pl.when(q < (NCH - 1) // 2 - 1)
        def _():
            stage_a(t0 + 3, 1)

        return carry

    lax.fori_loop(0, (NCH - 1) // 2, pair, 0)
    stage_b(NCH - 1, 0, jnp.bool_(True))
    drain_c(0)
    drain_c(1)


def _sc_pair(xap64, pj, pi):
    mesh = plsc.VectorSubcoreMesh(core_axis_name="c", subcore_axis_name="s")
    fn = functools.partial(
        pl.kernel,
        mesh=mesh,
        compiler_params=pltpu.CompilerParams(use_tc_tiling_on_sc=False),
        out_type=jax.ShapeDtypeStruct((E, DXAP), jnp.bfloat16),
        scratch_types=[
            [pltpu.VMEM((BC,), jnp.int32)] * 2,
            [pltpu.VMEM((BC,), jnp.int32)] * 2,
            [pltpu.VMEM((BC, DXAP), jnp.bfloat16)] * 2,
            [pltpu.VMEM((BC, DXAP), jnp.bfloat16)] * 2,
            [pltpu.VMEM((BC, DXAP), jnp.bfloat16)] * 2,
            [pltpu.SemaphoreType.DMA] * 2,
            [pltpu.SemaphoreType.DMA] * 2,
            [pltpu.SemaphoreType.DMA] * 2,
        ],
    )(_sc_pair_body)
    return fn(xap64, pj, pi)


def _sc_atom_body(xpa_hbm, u_hbm, pj_hbm, pi_hbm, zeros_hbm,
                  s_hbm,
                  idxj_v, idxis_v, xpj_v, u_v, accum, semA, semE):
    c = lax.axis_index("c")
    s = lax.axis_index("s")
    wid = c * NS + s
    base = wid * EPW

    pltpu.sync_copy(zeros_hbm.at[pl.ds(s * ROWS_PER_TILE, ROWS_PER_TILE)],
                    accum.at[pl.ds(s * ROWS_PER_TILE, ROWS_PER_TILE)])
    plsc.subcore_barrier()

    def stage_a(t, b):
        eb = base + t * BC
        pltpu.sync_copy(pj_hbm.at[pl.ds(eb, BC)], idxj_v[b])
        pltpu.async_copy(xpa_hbm.at[idxj_v[b]], xpj_v[b], semA[b])
        for k in range(NSC):
            pltpu.async_copy(pi_hbm.at[pl.ds(eb + k * SCB, SCB)], idxis_v[b].at[k], semE[b])
        pltpu.async_copy(u_hbm.at[pl.ds(eb, BC)], u_v[b], semE[b])

    def drain_a(b):
        pltpu.make_async_copy(xpa_hbm.at[idxj_v[b]], xpj_v[b], semA[b]).wait()
        for k in range(NSC):
            pltpu.make_async_copy(pi_hbm.at[pl.ds(0, SCB)], idxis_v[b].at[k], semE[b]).wait()
        pltpu.make_async_copy(u_hbm.at[pl.ds(0, BC)], u_v[b], semE[b]).wait()

    def stage_b(t, b):
        drain_a(b)

        def row(r2, rc):
            for rr in range(2):
                r = r2 * 2 + rr
                for cc in range(2):
                    sl = pl.ds(cc * 16, 16)
                    u_v[b][r, sl] = jnp.maximum(u_v[b][r, sl] + xpj_v[b][r, sl], 0.0)
            return rc

        lax.fori_loop(0, BC // 2, row, 0)
        for k in range(NSC):
            pltpu.sync_copy(u_v[b].at[pl.ds(k * SCB, SCB)],
                            accum.at[idxis_v[b].at[k]], add=True)

    stage_a(0, 0)
    stage_a(1, 1)

    def pair(q, carry):
        t0 = 2 * q
        stage_b(t0, 0)
        stage_a(t0 + 2, 0)
        stage_b(t0 + 1, 1)

        @pl.when(q < (NCH - 1) // 2 - 1)
        def _():
            stage_a(t0 + 3, 1)

        return carry

    lax.fori_loop(0, (NCH - 1) // 2, pair, 0)
    stage_b(NCH - 1, 0)

    plsc.subcore_barrier()
    pltpu.sync_copy(accum.at[pl.ds(s * ROWS_PER_TILE, ROWS_PER_TILE)],
                    s_hbm.at[pl.ds(c * NPAD + s * ROWS_PER_TILE, ROWS_PER_TILE)])


def _sc_atom(xpa, u, pj, pi, zeros):
    mesh = plsc.VectorSubcoreMesh(core_axis_name="c", subcore_axis_name="s")
    fn = functools.partial(
        pl.kernel,
        mesh=mesh,
        compiler_params=pltpu.CompilerParams(use_tc_tiling_on_sc=False),
        out_type=jax.ShapeDtypeStruct((NC * NPAD, DG), jnp.float32),
        scratch_types=[
            [pltpu.VMEM((BC,), jnp.int32)] * 2,
            [pltpu.VMEM((NSC, SCB), jnp.int32)] * 2,
            [pltpu.VMEM((BC, DG), jnp.float32)] * 2,
            [pltpu.VMEM((BC, DG), jnp.float32)] * 2,
            pltpu.VMEM_SHARED((NPAD, DG), jnp.float32),
            [pltpu.SemaphoreType.DMA] * 2,
            [pltpu.SemaphoreType.DMA] * 2,
        ],
    )(_sc_atom_body)
    return fn(xpa, u, pj, pi, zeros)


def _atom_post_body(s0_ref, s1_ref, pre_ref, aaa_ref, w_ref, out_ref):
    seg = s0_ref[...] + s1_ref[...]
    a_pa = jnp.maximum(pre_ref[...] + jnp.dot(seg, w_ref[...], preferred_element_type=jnp.float32), 0.0)
    out_ref[...] = jnp.maximum(a_pa + aaa_ref[...], 0.0)


def _atom_post(s0, s1, pre, aaa, wao_agg):
    bm = 2000
    return pl.pallas_call(
        _atom_post_body,
        grid=(N // bm,),
        in_specs=[
            pl.BlockSpec((bm, DG), lambda i: (i, 0)),
            pl.BlockSpec((bm, DG), lambda i: (i, 0)),
            pl.BlockSpec((bm, DO), lambda i: (i, 0)),
            pl.BlockSpec((bm, DO), lambda i: (i, 0)),
            pl.BlockSpec((DG, DO), lambda i: (0, 0)),
        ],
        out_specs=pl.BlockSpec((bm, DO), lambda i: (i, 0)),
        out_shape=jax.ShapeDtypeStruct((N, DO), jnp.float32),
    )(s0, s1, pre, aaa, wao_agg)


def _pair_post_body(pf_ref, g2_ref, wap_ref, bap_ref, wpp_ref, bpp_ref, out_ref):
    pf = pf_ref[...]
    papa = jnp.maximum(
        jnp.dot(pf, wap_ref[...], preferred_element_type=jnp.float32)
        + g2_ref[:, :DO].astype(jnp.float32) + bap_ref[...], 0.0)
    ppp = jnp.maximum(
        jnp.dot(pf, wpp_ref[...], preferred_element_type=jnp.float32) + bpp_ref[...], 0.0)
    out_ref[...] = jnp.maximum(papa + ppp, 0.0)


def _pair_post(pf, g2, wap_p, b_ap, wpp, b_pp):
    bm = 8000
    return pl.pallas_call(
        _pair_post_body,
        grid=(E // bm,),
        in_specs=[
            pl.BlockSpec((bm, DP), lambda i: (i, 0)),
            pl.BlockSpec((bm, DXAP), lambda i: (i, 0)),
            pl.BlockSpec((DP, DO), lambda i: (0, 0)),
            pl.BlockSpec((1, DO), lambda i: (0, 0)),
            pl.BlockSpec((DP, DO), lambda i: (0, 0)),
            pl.BlockSpec((1, DO), lambda i: (0, 0)),
        ],
        out_specs=pl.BlockSpec((bm, DO), lambda i: (i, 0)),
        out_shape=jax.ShapeDtypeStruct((E, DO), jnp.float32),
    )(pf, g2, wap_p, b_ap, wpp, b_pp)


def kernel(atom_features, pair_features, pair_split, atom_to_pair,
           W_pa, b_pa, W_ao, b_ao, W_aa, b_aa, W_ap, b_ap, W_pp, b_pp):
    pi = atom_to_pair[:, 0]
    pj = atom_to_pair[:, 1]

    wpa_p, wpa_a = W_pa[:DP], W_pa[DP:]
    wap_p, wap_a = W_ap[:DP], W_ap[DP:]
    wao_top, wao_agg = W_ao[:DA], W_ao[DA:]

    b_pa2 = b_pa.reshape(1, DG)
    b_ao2 = b_ao.reshape(1, DO)
    b_aa2 = b_aa.reshape(1, DO)
    b_ap2 = b_ap.reshape(1, DO)
    b_pp2 = b_pp.reshape(1, DO)

    xpa, xap64, pre, aaa = _node_pre(atom_features, wpa_a, wap_a, wao_top, b_ao2, waa=W_aa, b_aa=b_aa2)
    u = _edge_pre(pair_features, wpa_p, b_pa2)
    zeros = jnp.zeros((NPAD, DG), jnp.float32)
    g2 = _sc_pair(xap64, pj, pi)
    s_all = _sc_atom(xpa, u, pj, pi, zeros)
    s0 = lax.slice(s_all, (0, 0), (N, DG))
    s1 = lax.slice(s_all, (NPAD, 0), (NPAD + N, DG))
    atom_hidden = _atom_post(s0, s1, pre, aaa, wao_agg)
    pair_hidden = _pair_post(pair_features, g2, wap_p, b_ap2, wpp=W_pp, b_pp=b_pp2)
    return (atom_hidden, pair_hidden)


# speedup vs baseline: 2.3706x; 1.0052x over previous
import functools

import jax
import jax.numpy as jnp
from jax import lax
from jax.experimental import pallas as pl
from jax.experimental.pallas import tpu as pltpu
from jax.experimental.pallas import tpu_sc as plsc

N = 50000
E = 800000
DA = 75
DP = 16
DG = 32
DO = 50
DXAP = 64
DXC = 96

NC = 2
NS = 16
NW = NC * NS
EPW = E // NW
BC = 200
NCH = EPW // BC
SCB = 40
NSC = BC // SCB
NPAD = 50048
ROWS_PER_TILE = NPAD // NS
ZROWS = 184
ZREP = ROWS_PER_TILE // ZROWS


def _node_pre_body(a_ref, wpa_ref, wap_ref, waot_ref, bao_ref, waa_ref, baa_ref,
                   xpa_ref, xap_ref, pre_ref, aaa_ref):
    a = a_ref[...]
    xap = jnp.dot(a, wap_ref[...], preferred_element_type=jnp.float32)
    xpa_ref[...] = jnp.dot(a, wpa_ref[...], preferred_element_type=jnp.float32)
    pad = jnp.zeros((a.shape[0], DXAP - DO), jnp.float32)
    xap_ref[...] = jnp.concatenate([xap, pad], axis=1).astype(jnp.bfloat16)
    pre_ref[...] = jnp.dot(a, waot_ref[...], preferred_element_type=jnp.float32) + bao_ref[...]
    aaa_ref[...] = jnp.maximum(jnp.dot(a, waa_ref[...], preferred_element_type=jnp.float32) + baa_ref[...], 0.0)


def _node_pre(atom, wpa_a, wap_a, wao_top, b_ao, waa, b_aa):
    bm = 2000
    grid = (N // bm,)
    return pl.pallas_call(
        _node_pre_body,
        grid=grid,
        in_specs=[
            pl.BlockSpec((bm, DA), lambda i: (i, 0)),
            pl.BlockSpec((DA, DG), lambda i: (0, 0)),
            pl.BlockSpec((DA, DO), lambda i: (0, 0)),
            pl.BlockSpec((DA, DO), lambda i: (0, 0)),
            pl.BlockSpec((1, DO), lambda i: (0, 0)),
            pl.BlockSpec((DA, DO), lambda i: (0, 0)),
            pl.BlockSpec((1, DO), lambda i: (0, 0)),
        ],
        out_specs=[
            pl.BlockSpec((bm, DG), lambda i: (i, 0)),
            pl.BlockSpec((bm, DXAP), lambda i: (i, 0)),
            pl.BlockSpec((bm, DO), lambda i: (i, 0)),
            pl.BlockSpec((bm, DO), lambda i: (i, 0)),
        ],
        out_shape=[
            jax.ShapeDtypeStruct((N, DG), jnp.float32),
            jax.ShapeDtypeStruct((N, DXAP), jnp.bfloat16),
            jax.ShapeDtypeStruct((N, DO), jnp.float32),
            jax.ShapeDtypeStruct((N, DO), jnp.float32),
        ],
    )(atom, wpa_a, wap_a, wao_top, b_ao, waa, b_aa)


def _edge_pre_body(pf_ref, w_ref, b_ref, u_ref):
    u_ref[...] = jnp.dot(pf_ref[...], w_ref[...], preferred_element_type=jnp.float32) + b_ref[...]


def _edge_pre(pf, wpa_p, b_pa):
    bm = 8000
    return pl.pallas_call(
        _edge_pre_body,
        grid=(E // bm,),
        in_specs=[
            pl.BlockSpec((bm, DP), lambda i: (i, 0)),
            pl.BlockSpec((DP, DG), lambda i: (0, 0)),
            pl.BlockSpec((1, DG), lambda i: (0, 0)),
        ],
        out_specs=pl.BlockSpec((bm, DG), lambda i: (i, 0)),
        out_shape=jax.ShapeDtypeStruct((E, DG), jnp.float32),
    )(pf, wpa_p, b_pa)


BCP = 1000
NCHP = EPW // BCP


def _sc_pair_body(xap_hbm, pj_hbm, pi_hbm, g2_hbm,
                  idxj_v, idxi_v, vi_v, g2b_v, semA, semB, semC):
    c = lax.axis_index("c")
    s = lax.axis_index("s")
    wid = c * NS + s
    base = wid * EPW

    def drain_c(b):
        pltpu.make_async_copy(g2b_v[b], g2_hbm.at[pl.ds(0, BCP)], semC[b]).wait()

    def load_and_gather(t, b, drain_pred):
        @pl.when(drain_pred)
        def _():
            drain_c(b)

        eb = base + t * BCP
        pltpu.sync_copy(pj_hbm.at[pl.ds(eb, BCP)], idxj_v)
        pltpu.sync_copy(pi_hbm.at[pl.ds(eb, BCP)], idxi_v)
        pltpu.async_copy(xap_hbm.at[idxj_v], g2b_v[b], semA[b])
        pltpu.async_copy(xap_hbm.at[idxi_v], vi_v[b], semB[b])

    def drain_gathers(b):
        pltpu.make_async_copy(xap_hbm.at[idxj_v], g2b_v[b], semA[b]).wait()
        pltpu.make_async_copy(xap_hbm.at[idxi_v], vi_v[b], semB[b]).wait()

    def stage_b(t, b, more, drain_pred):
        drain_gathers(b)

        @pl.when(more)
        def _():
            load_and_gather(t + 1, 1 - b, drain_pred)

        def row(r2, rc):
            for rr in range(2):
                r = r2 * 2 + rr
                for cc in range(2):
                    sl = pl.ds(cc * 32, 32)
                    g2b_v[b][r, sl] = g2b_v[b][r, sl] + vi_v[b][r, sl]
            return rc

        lax.fori_loop(0, BCP // 2, row, 0)
        eb = base + t * BCP
        pltpu.async_copy(g2b_v[b], g2_hbm.at[pl.ds(eb, BCP)], semC[b])

    load_and_gather(0, 0, jnp.bool_(False))

    def pair(q, carry):
        t0 = 2 * q
        stage_b(t0, 0, jnp.bool_(True), q > 0)
        stage_b(t0 + 1, 1, jnp.bool_(True), q > 0)
        return carry

    lax.fori_loop(0, NCHP // 2, pair, 0)
    stage_b(NCHP - 1, 0, jnp.bool_(False), jnp.bool_(False))
    drain_c(0)
    drain_c(1)


def _sc_pair(xap64, pj, pi):
    mesh = plsc.VectorSubcoreMesh(core_axis_name="c", subcore_axis_name="s")
    fn = functools.partial(
        pl.kernel,
        mesh=mesh,
        compiler_params=pltpu.CompilerParams(use_tc_tiling_on_sc=False),
        out_type=jax.ShapeDtypeStruct((E, DXAP), jnp.bfloat16),
        scratch_types=[
            pltpu.VMEM((BCP,), jnp.int32),
            pltpu.VMEM((BCP,), jnp.int32),
            [pltpu.VMEM((BCP, DXAP), jnp.bfloat16)] * 2,
            [pltpu.VMEM((BCP, DXAP), jnp.bfloat16)] * 2,
            [pltpu.SemaphoreType.DMA] * 2,
            [pltpu.SemaphoreType.DMA] * 2,
            [pltpu.SemaphoreType.DMA] * 2,
        ],
    )(_sc_pair_body)
    return fn(xap64, pj, pi)


SCA = 104
SCB2 = BC - SCA


def _sc_atom_body(xpa_hbm, u_hbm, pj_hbm, pi_hbm, zeros_hbm,
                  s_hbm,
                  idxj_v, idxa_v, idxb_v, xpj_v, u_v, accum, semA, semE, semD):
    c = lax.axis_index("c")
    s = lax.axis_index("s")
    wid = c * NS + s
    base = wid * EPW

    pltpu.sync_copy(zeros_hbm.at[pl.ds(s * ROWS_PER_TILE, ROWS_PER_TILE)],
                    accum.at[pl.ds(s * ROWS_PER_TILE, ROWS_PER_TILE)])
    plsc.subcore_barrier()

    def drain_scatters(b):
        pltpu.make_async_copy(u_v[b].at[pl.ds(0, SCA)],
                              accum.at[idxa_v[b]], semD[b]).wait()
        pltpu.make_async_copy(u_v[b].at[pl.ds(SCA, SCB2)],
                              accum.at[idxb_v[b]], semD[b]).wait()

    def load_and_gather(t, b, drain_pred):
        @pl.when(drain_pred)
        def _():
            drain_scatters(b)

        eb = base + t * BC
        pltpu.sync_copy(pj_hbm.at[pl.ds(eb, BC)], idxj_v)
        pltpu.async_copy(xpa_hbm.at[idxj_v], xpj_v[b], semA[b])
        pltpu.async_copy(pi_hbm.at[pl.ds(eb, SCA)], idxa_v[b], semE[b])
        pltpu.async_copy(pi_hbm.at[pl.ds(eb + SCA, SCB2)], idxb_v[b], semE[b])
        pltpu.async_copy(u_hbm.at[pl.ds(eb, BC)], u_v[b], semE[b])

    def drain_loads(b):
        pltpu.make_async_copy(xpa_hbm.at[idxj_v], xpj_v[b], semA[b]).wait()
        pltpu.make_async_copy(pi_hbm.at[pl.ds(0, SCA)], idxa_v[b], semE[b]).wait()
        pltpu.make_async_copy(pi_hbm.at[pl.ds(0, SCB2)], idxb_v[b], semE[b]).wait()
        pltpu.make_async_copy(u_hbm.at[pl.ds(0, BC)], u_v[b], semE[b]).wait()

    def stage_b(t, b, more, drain_pred):
        drain_loads(b)

        @pl.when(more)
        def _():
            load_and_gather(t + 1, 1 - b, drain_pred)

        def row(r2, rc):
            for rr in range(2):
                r = r2 * 2 + rr
                for cc in range(2):
                    sl = pl.ds(cc * 16, 16)
                    u_v[b][r, sl] = jnp.maximum(u_v[b][r, sl] + xpj_v[b][r, sl], 0.0)
            return rc

        lax.fori_loop(0, BC // 2, row, 0)
        pltpu.async_copy(u_v[b].at[pl.ds(0, SCA)],
                         accum.at[idxa_v[b]], semD[b], add=True)
        pltpu.async_copy(u_v[b].at[pl.ds(SCA, SCB2)],
                         accum.at[idxb_v[b]], semD[b], add=True)

    load_and_gather(0, 0, jnp.bool_(False))

    def pair(q, carry):
        t0 = 2 * q
        stage_b(t0, 0, jnp.bool_(True), q > 0)
        stage_b(t0 + 1, 1, jnp.bool_(True), jnp.bool_(True))
        return carry

    lax.fori_loop(0, (NCH - 1) // 2, pair, 0)
    stage_b(NCH - 1, 0, jnp.bool_(False), jnp.bool_(False))
    drain_scatters(0)
    drain_scatters(1)

    plsc.subcore_barrier()
    pltpu.sync_copy(accum.at[pl.ds(s * ROWS_PER_TILE, ROWS_PER_TILE)],
                    s_hbm.at[pl.ds(c * NPAD + s * ROWS_PER_TILE, ROWS_PER_TILE)])


def _sc_atom(xpa, u, pj, pi, zeros):
    mesh = plsc.VectorSubcoreMesh(core_axis_name="c", subcore_axis_name="s")
    fn = functools.partial(
        pl.kernel,
        mesh=mesh,
        compiler_params=pltpu.CompilerParams(use_tc_tiling_on_sc=False),
        out_type=jax.ShapeDtypeStruct((NC * NPAD, DG), jnp.float32),
        scratch_types=[
            pltpu.VMEM((BC,), jnp.int32),
            [pltpu.VMEM((SCA,), jnp.int32)] * 2,
            [pltpu.VMEM((SCB2,), jnp.int32)] * 2,
            [pltpu.VMEM((BC, DG), jnp.float32)] * 2,
            [pltpu.VMEM((BC, DG), jnp.float32)] * 2,
            pltpu.VMEM_SHARED((NPAD, DG), jnp.float32),
            [pltpu.SemaphoreType.DMA] * 2,
            [pltpu.SemaphoreType.DMA] * 2,
            [pltpu.SemaphoreType.DMA] * 2,
        ],
    )(_sc_atom_body)
    return fn(xpa, u, pj, pi, zeros)


def _atom_post_body(s0_ref, s1_ref, pre_ref, aaa_ref, w_ref, out_ref):
    seg = s0_ref[...] + s1_ref[...]
    a_pa = jnp.maximum(pre_ref[...] + jnp.dot(seg, w_ref[...], preferred_element_type=jnp.float32), 0.0)
    out_ref[...] = jnp.maximum(a_pa + aaa_ref[...], 0.0)


def _atom_post(s0, s1, pre, aaa, wao_agg):
    bm = 2000
    return pl.pallas_call(
        _atom_post_body,
        grid=(N // bm,),
        in_specs=[
            pl.BlockSpec((bm, DG), lambda i: (i, 0)),
            pl.BlockSpec((bm, DG), lambda i: (i, 0)),
            pl.BlockSpec((bm, DO), lambda i: (i, 0)),
            pl.BlockSpec((bm, DO), lambda i: (i, 0)),
            pl.BlockSpec((DG, DO), lambda i: (0, 0)),
        ],
        out_specs=pl.BlockSpec((bm, DO), lambda i: (i, 0)),
        out_shape=jax.ShapeDtypeStruct((N, DO), jnp.float32),
    )(s0, s1, pre, aaa, wao_agg)


def _pair_post_body(pf_ref, g2_ref, wap_ref, bap_ref, wpp_ref, bpp_ref, out_ref):
    pf = pf_ref[...]
    papa = jnp.maximum(
        jnp.dot(pf, wap_ref[...], preferred_element_type=jnp.float32)
        + g2_ref[:, :DO].astype(jnp.float32) + bap_ref[...], 0.0)
    ppp = jnp.maximum(
        jnp.dot(pf, wpp_ref[...], preferred_element_type=jnp.float32) + bpp_ref[...], 0.0)
    out_ref[...] = jnp.maximum(papa + ppp, 0.0)


def _pair_post(pf, g2, wap_p, b_ap, wpp, b_pp):
    bm = 8000
    return pl.pallas_call(
        _pair_post_body,
        grid=(E // bm,),
        in_specs=[
            pl.BlockSpec((bm, DP), lambda i: (i, 0)),
            pl.BlockSpec((bm, DXAP), lambda i: (i, 0)),
            pl.BlockSpec((DP, DO), lambda i: (0, 0)),
            pl.BlockSpec((1, DO), lambda i: (0, 0)),
            pl.BlockSpec((DP, DO), lambda i: (0, 0)),
            pl.BlockSpec((1, DO), lambda i: (0, 0)),
        ],
        out_specs=pl.BlockSpec((bm, DO), lambda i: (i, 0)),
        out_shape=jax.ShapeDtypeStruct((E, DO), jnp.float32),
    )(pf, g2, wap_p, b_ap, wpp, b_pp)


def kernel(atom_features, pair_features, pair_split, atom_to_pair,
           W_pa, b_pa, W_ao, b_ao, W_aa, b_aa, W_ap, b_ap, W_pp, b_pp):
    pi = atom_to_pair[:, 0]
    pj = atom_to_pair[:, 1]

    wpa_p, wpa_a = W_pa[:DP], W_pa[DP:]
    wap_p, wap_a = W_ap[:DP], W_ap[DP:]
    wao_top, wao_agg = W_ao[:DA], W_ao[DA:]

    b_pa2 = b_pa.reshape(1, DG)
    b_ao2 = b_ao.reshape(1, DO)
    b_aa2 = b_aa.reshape(1, DO)
    b_ap2 = b_ap.reshape(1, DO)
    b_pp2 = b_pp.reshape(1, DO)

    xpa, xap64, pre, aaa = _node_pre(atom_features, wpa_a, wap_a, wao_top, b_ao2, waa=W_aa, b_aa=b_aa2)
    u = _edge_pre(pair_features, wpa_p, b_pa2)
    zeros = jnp.zeros((NPAD, DG), jnp.float32)
    g2 = _sc_pair(xap64, pj, pi)
    s_all = _sc_atom(xpa, u, pj, pi, zeros)
    s0 = lax.slice(s_all, (0, 0), (N, DG))
    s1 = lax.slice(s_all, (NPAD, 0), (NPAD + N, DG))
    atom_hidden = _atom_post(s0, s1, pre, aaa, wao_agg)
    pair_hidden = _pair_post(pair_features, g2, wap_p, b_ap2, wpp=W_pp, b_pp=b_pp2)
    return (atom_hidden, pair_hidden)
